# baseline (speedup 1.0000x reference)
import functools

import jax
import jax.numpy as jnp
from jax import lax
from jax.experimental import pallas as pl
from jax.experimental.pallas import tpu as pltpu
from jax.experimental.pallas import tpu_sc as plsc

NC, NS = 2, 16
NW = NC * NS
IDX_MINOR = 128
GPC = 4
CHUNK = IDX_MINOR * GPC


def kernel(x, W):
    B, L = x.shape
    V, E = W.shape
    N = B * L
    assert N % (NW * IDX_MINOR) == 0
    per_w = N // NW
    rows_per_w = per_w // IDX_MINOR
    nchunks = per_w // CHUNK
    assert nchunks * CHUNK == per_w

    x2d = x.reshape(N // IDX_MINOR, IDX_MINOR).astype(jnp.int32)

    mesh = plsc.VectorSubcoreMesh(core_axis_name="c", subcore_axis_name="s")

    @functools.partial(
        pl.kernel,
        out_type=jax.ShapeDtypeStruct((N, E), jnp.float32),
        mesh=mesh,
        scratch_types=[
            pltpu.VMEM((rows_per_w, IDX_MINOR), jnp.int32),
            pltpu.VMEM((CHUNK, E), jnp.float32),
            pltpu.SemaphoreType.DMA,
        ],
        compiler_params=pltpu.CompilerParams(use_tc_tiling_on_sc=False),
    )
    def emb(x_hbm, w_hbm, out_hbm, idx_v, buf, gsem):
        wid = lax.axis_index("s") * NC + lax.axis_index("c")
        row0 = wid * rows_per_w
        base = wid * per_w
        pltpu.sync_copy(x_hbm.at[pl.ds(row0, rows_per_w)], idx_v)

        def body(g, carry):
            copies = [
                pltpu.async_copy(
                    w_hbm.at[idx_v.at[g * GPC + j]],
                    buf.at[pl.ds(j * IDX_MINOR, IDX_MINOR)],
                    gsem,
                )
                for j in range(GPC)
            ]
            for c in copies:
                c.wait()
            pltpu.sync_copy(buf, out_hbm.at[pl.ds(base + g * CHUNK, CHUNK)])
            return carry

        lax.fori_loop(0, nchunks, body, 0)

    out = emb(x2d, W)
    return out.reshape(B, L, E)


# speedup vs baseline: 1.0209x; 1.0209x over previous
import functools

import jax
import jax.numpy as jnp
from jax import lax
from jax.experimental import pallas as pl
from jax.experimental.pallas import tpu as pltpu
from jax.experimental.pallas import tpu_sc as plsc

NC, NS = 2, 16
NW = NC * NS
IDX_MINOR = 128
GPC = 5
CHUNK = IDX_MINOR * GPC


def kernel(x, W):
    B, L = x.shape
    V, E = W.shape
    N = B * L
    assert N % (NW * IDX_MINOR) == 0
    per_w = N // NW
    rows_per_w = per_w // IDX_MINOR
    nchunks = per_w // CHUNK
    assert nchunks * CHUNK == per_w and nchunks % 2 == 0
    npairs = nchunks // 2

    x2d = x.reshape(N // IDX_MINOR, IDX_MINOR).astype(jnp.int32)

    mesh = plsc.VectorSubcoreMesh(core_axis_name="c", subcore_axis_name="s")

    @functools.partial(
        pl.kernel,
        out_type=jax.ShapeDtypeStruct((N, E), jnp.float32),
        mesh=mesh,
        scratch_types=[
            pltpu.VMEM((rows_per_w, IDX_MINOR), jnp.int32),
            pltpu.VMEM((CHUNK, E), jnp.float32),
            pltpu.VMEM((CHUNK, E), jnp.float32),
            pltpu.SemaphoreType.DMA,
            pltpu.SemaphoreType.DMA,
            pltpu.SemaphoreType.DMA,
            pltpu.SemaphoreType.DMA,
        ],
        compiler_params=pltpu.CompilerParams(use_tc_tiling_on_sc=False),
    )
    def emb(x_hbm, w_hbm, out_hbm, idx_v, buf0, buf1, gsem0, gsem1, ssem0,
            ssem1):
        wid = lax.axis_index("s") * NC + lax.axis_index("c")
        row0 = wid * rows_per_w
        base = wid * per_w
        pltpu.sync_copy(x_hbm.at[pl.ds(row0, rows_per_w)], idx_v)

        bufs = (buf0, buf1)
        gsems = (gsem0, gsem1)
        ssems = (ssem0, ssem1)

        def issue_gathers(g, b):
            for j in range(GPC):
                pltpu.async_copy(
                    w_hbm.at[idx_v.at[g * GPC + j]],
                    bufs[b].at[pl.ds(j * IDX_MINOR, IDX_MINOR)],
                    gsems[b],
                )

        def wait_gathers(b):
            pltpu.make_async_copy(
                out_hbm.at[pl.ds(base, CHUNK)], bufs[b], gsems[b]
            ).wait()

        def issue_store(g, b):
            pltpu.async_copy(
                bufs[b], out_hbm.at[pl.ds(base + g * CHUNK, CHUNK)], ssems[b]
            )

        def wait_store(b):
            pltpu.make_async_copy(
                bufs[b], out_hbm.at[pl.ds(base, CHUNK)], ssems[b]
            ).wait()

        issue_gathers(0, 0)
        issue_gathers(1, 1)

        def body(i, carry):
            g0 = 2 * i
            wait_gathers(0)
            issue_store(g0, 0)
            wait_gathers(1)
            issue_store(g0 + 1, 1)
            wait_store(0)
            issue_gathers(g0 + 2, 0)
            wait_store(1)
            issue_gathers(g0 + 3, 1)
            return carry

        lax.fori_loop(0, npairs - 1, body, 0)

        g0 = nchunks - 2
        wait_gathers(0)
        issue_store(g0, 0)
        wait_gathers(1)
        issue_store(g0 + 1, 1)
        wait_store(0)
        wait_store(1)

    out = emb(x2d, W)
    return out.reshape(B, L, E)
